# baseline (speedup 1.0000x reference)
import functools

import jax
import jax.numpy as jnp
from jax import lax
from jax.experimental import pallas as pl
from jax.experimental.pallas import tpu as pltpu
from jax.experimental.pallas import tpu_sc as plsc

N = 10000
E = 320000
D = 128
HIM = 16
HEADS = 8
OUT = 64
NEG_SLOPE = 0.2

NPAD = 10016
NCORES = 2
NSUB = 16
NW = NCORES * NSUB
K = 96
EPAD = 331776
CPW = EPAD // (NW * K)
RPW = NPAD // NSUB
NEG = -1e30


def _seg_matrix(heads, ch):
    r = lax.broadcasted_iota(jnp.int32, (heads * ch, heads), 0) // ch
    c = lax.broadcasted_iota(jnp.int32, (heads * ch, heads), 1)
    return (r == c).astype(jnp.float32)


def _prep1_body(x_ref, w_ref, asrc_ref, adst_ref, aug_ref, dstt_ref, bvec_ref):
    h = jnp.dot(x_ref[...], w_ref[...], preferred_element_type=jnp.float32)
    seg = _seg_matrix(HEADS, HIM)
    asrc = jnp.dot(h * asrc_ref[...], seg, preferred_element_type=jnp.float32)
    adst = jnp.dot(h * adst_ref[...], seg, preferred_element_type=jnp.float32)
    bsum = (jnp.max(asrc, axis=0, keepdims=True)
            + jnp.max(adst, axis=0, keepdims=True))
    bvec_ref[...] = jnp.concatenate(
        [bsum, jnp.zeros((1, 8), jnp.float32)], axis=1)
    aug_ref[...] = jnp.concatenate(
        [h, asrc, jnp.full((N, 8), NEG, jnp.float32)], axis=1)
    dstt = jnp.concatenate([adst, jnp.full((N, 8), NEG, jnp.float32)], axis=1)
    dstt_ref[...] = jnp.concatenate(
        [dstt, jnp.full((NPAD - N, 16), NEG, jnp.float32)], axis=0)


_prep1 = pl.pallas_call(
    _prep1_body,
    out_shape=[
        jax.ShapeDtypeStruct((N, 144), jnp.float32),
        jax.ShapeDtypeStruct((NPAD, 16), jnp.float32),
        jax.ShapeDtypeStruct((1, 16), jnp.float32),
    ],
)


def _prep2_body(parts_ref, b1_ref, w2_ref, asrc2_ref, adst2_ref,
                aug_ref, dstt_ref, bvec_ref):
    aug1 = parts_ref[0] + parts_ref[1]
    msg = aug1[:N, 0:128]
    wsum = aug1[:N, 128:136]
    denf = jnp.dot(wsum, _seg_matrix(HEADS, HIM).T,
                   preferred_element_type=jnp.float32)
    x2 = jnp.maximum(msg / denf + b1_ref[...], 0.0)
    h2 = jnp.dot(x2, w2_ref[...], preferred_element_type=jnp.float32)

    lane0 = (lax.broadcasted_iota(jnp.int32, (1, 16), 1) == 0)
    a_s = asrc2_ref[...].T * lane0.astype(jnp.float32)
    a_d = adst2_ref[...].T * lane0.astype(jnp.float32)
    att_s = jnp.dot(h2, a_s, preferred_element_type=jnp.float32)
    att_d = jnp.dot(h2, a_d, preferred_element_type=jnp.float32)
    bsum = (jnp.max(att_s[:, 0:1], axis=0, keepdims=True)
            + jnp.max(att_d[:, 0:1], axis=0, keepdims=True))
    bvec_ref[...] = jnp.where(lane0, bsum, 0.0)
    att_s = jnp.where(lane0, att_s, NEG)
    att_d = jnp.where(lane0, att_d, NEG)
    aug_ref[...] = jnp.concatenate([h2, att_s], axis=1)
    dstt_ref[...] = jnp.concatenate(
        [att_d, jnp.full((NPAD - N, 16), NEG, jnp.float32)], axis=0)


_prep2 = pl.pallas_call(
    _prep2_body,
    out_shape=[
        jax.ShapeDtypeStruct((N, 80), jnp.float32),
        jax.ShapeDtypeStruct((NPAD, 16), jnp.float32),
        jax.ShapeDtypeStruct((1, 16), jnp.float32),
    ],
)


def _final_body(parts_ref, b2_ref, out_ref):
    aug2 = parts_ref[0] + parts_ref[1]
    msg = aug2[:N, 0:64]
    den = aug2[:N, 64:65]
    out_ref[...] = msg / den + b2_ref[...]


_final = pl.pallas_call(
    _final_body,
    out_shape=jax.ShapeDtypeStruct((N, OUT), jnp.float32),
)


def _edge_pass_body(row, hc, heads, ch,
                    aug_hbm, dstt_hbm, edges_hbm, bvec_hbm, out_hbm,
                    idx_a, idx_b, rows_a, rows_b, drows_a, drows_b,
                    bvec, acc, sem_a1, sem_a2, sem_b1, sem_b2,
                    sem_ia, sem_ib):
    cid = lax.axis_index("c")
    sid = lax.axis_index("s")
    wid = sid * NCORES + cid

    def _zero_row(i, _):
        for j in range(row // 16):
            rows_a[i, pl.ds(j * 16, 16)] = jnp.zeros((16,), jnp.float32)
        return 0
    lax.fori_loop(0, K, _zero_row, 0)
    base_row = sid * RPW
    for t in range(RPW // K):
        pltpu.sync_copy(rows_a, acc.at[pl.ds(base_row + t * K, K)])
    rem = RPW % K
    if rem:
        pltpu.sync_copy(rows_a.at[pl.ds(0, rem)],
                        acc.at[pl.ds(base_row + (RPW // K) * K, rem)])
    plsc.subcore_barrier()

    pltpu.sync_copy(bvec_hbm, bvec)
    base_c = wid * CPW

    def _issue_gather(idx, rows, drows, s1, s2):
        pltpu.async_copy(aug_hbm.at[idx.at[0]], rows, s1)
        pltpu.async_copy(dstt_hbm.at[idx.at[1]], drows, s2)

    def _wait_gather(rows, drows, s1, s2):
        pltpu.make_async_copy(aug_hbm.at[idx_a.at[0]], rows, s1).wait()
        pltpu.make_async_copy(dstt_hbm.at[idx_a.at[1]], drows, s2).wait()

    def _wait_idx(idx, sem):
        pltpu.make_async_copy(edges_hbm.at[0], idx, sem).wait()

    def _compute_scatter(idx, rows, drows):
        bv = bvec[...]

        @plsc.parallel_loop(0, K, 1, unroll=4)
        def _edge(i):
            ev = rows[i, pl.ds(hc, 16)] + drows[i, :]
            ev = jnp.where(ev >= 0.0, ev, ev * NEG_SLOPE)
            wv = jnp.exp(ev - bv)
            rows[i, pl.ds(hc, 16)] = wv
            for j in range(heads):
                ws = wv[j]
                for v in range(ch // 16):
                    off = j * ch + v * 16
                    rows[i, pl.ds(off, 16)] = rows[i, pl.ds(off, 16)] * ws
        pltpu.sync_copy(rows, acc.at[idx.at[1]], add=True)

    pltpu.sync_copy(edges_hbm.at[base_c], idx_a)
    pltpu.async_copy(edges_hbm.at[base_c + 1], idx_b, sem_ib)
    _issue_gather(idx_a, rows_a, drows_a, sem_a1, sem_a2)

    def _pair(r, _):
        c = base_c + r * 2
        _wait_gather(rows_a, drows_a, sem_a1, sem_a2)
        _wait_idx(idx_b, sem_ib)
        _issue_gather(idx_b, rows_b, drows_b, sem_b1, sem_b2)
        _compute_scatter(idx_a, rows_a, drows_a)
        pltpu.async_copy(edges_hbm.at[c + 2], idx_a, sem_ia)
        _wait_gather(rows_b, drows_b, sem_b1, sem_b2)
        _wait_idx(idx_a, sem_ia)
        _issue_gather(idx_a, rows_a, drows_a, sem_a1, sem_a2)
        _compute_scatter(idx_b, rows_b, drows_b)
        pltpu.async_copy(edges_hbm.at[c + 3], idx_b, sem_ib)
        return 0
    lax.fori_loop(0, CPW // 2, _pair, 0)
    _wait_gather(rows_a, drows_a, sem_a1, sem_a2)
    _wait_idx(idx_b, sem_ib)

    plsc.subcore_barrier()
    pltpu.sync_copy(acc.at[pl.ds(base_row, RPW)],
                    out_hbm.at[cid].at[pl.ds(base_row, RPW)])


def _make_edge_pass(row, hc, heads, ch):
    return functools.partial(
        pl.kernel,
        out_type=jax.ShapeDtypeStruct((NCORES, NPAD, row), jnp.float32),
        mesh=plsc.VectorSubcoreMesh(core_axis_name="c", subcore_axis_name="s"),
        compiler_params=pltpu.CompilerParams(use_tc_tiling_on_sc=False),
        scratch_types=[
            pltpu.VMEM((2, K), jnp.int32),
            pltpu.VMEM((2, K), jnp.int32),
            pltpu.VMEM((K, row), jnp.float32),
            pltpu.VMEM((K, row), jnp.float32),
            pltpu.VMEM((K, 16), jnp.float32),
            pltpu.VMEM((K, 16), jnp.float32),
            pltpu.VMEM((16,), jnp.float32),
            pltpu.VMEM_SHARED((NPAD, row), jnp.float32),
            pltpu.SemaphoreType.DMA,
            pltpu.SemaphoreType.DMA,
            pltpu.SemaphoreType.DMA,
            pltpu.SemaphoreType.DMA,
            pltpu.SemaphoreType.DMA,
            pltpu.SemaphoreType.DMA,
        ],
    )(functools.partial(_edge_pass_body, row, hc, heads, ch))


_edge_pass1 = _make_edge_pass(144, 128, HEADS, HIM)
_edge_pass2 = _make_edge_pass(80, 64, 1, OUT)


def kernel(x, edge_index, W1, a_src1, a_dst1, b1, W2, a_src2, a_dst2, b2):
    loops = jnp.arange(N, dtype=jnp.int32)
    npad_e = EPAD + 2 * K - (E + N)
    src = jnp.concatenate(
        [edge_index[0], loops,
         jnp.zeros((npad_e,), jnp.int32)]).reshape(-1, K)
    dst = jnp.concatenate(
        [edge_index[1], loops,
         jnp.full((npad_e,), N, jnp.int32)]).reshape(-1, K)
    edges = jnp.stack([src, dst], axis=1)

    aug1, dstt1, bvec1 = _prep1(
        x, W1, a_src1.reshape(1, HEADS * HIM), a_dst1.reshape(1, HEADS * HIM))
    parts1 = _edge_pass1(aug1, dstt1, edges, bvec1.reshape(16))

    aug2, dstt2, bvec2 = _prep2(
        parts1, b1.reshape(1, HEADS * HIM), W2, a_src2, a_dst2)
    parts2 = _edge_pass2(aug2, dstt2, edges, bvec2.reshape(16))

    return _final(parts2, b2.reshape(1, OUT))


# speedup vs baseline: 1.1521x; 1.1521x over previous
import functools

import jax
import jax.numpy as jnp
from jax import lax
from jax.experimental import pallas as pl
from jax.experimental.pallas import tpu as pltpu
from jax.experimental.pallas import tpu_sc as plsc

N = 10000
E = 320000
D = 128
HIM = 16
HEADS = 8
OUT = 64
NEG_SLOPE = 0.2

NPAD = 10016
NCORES = 2
NSUB = 16
NW = NCORES * NSUB
K = 80
EPAD = 330240
CPW = EPAD // (NW * K)
RPW = NPAD // NSUB
NEG = -1e30


def _seg_matrix(heads, ch):
    r = lax.broadcasted_iota(jnp.int32, (heads * ch, heads), 0) // ch
    c = lax.broadcasted_iota(jnp.int32, (heads * ch, heads), 1)
    return (r == c).astype(jnp.float32)


def _prep1_body(x_ref, w_ref, asrc_ref, adst_ref, aug_ref, dstt_ref, bvec_ref):
    h = jnp.dot(x_ref[...], w_ref[...], preferred_element_type=jnp.float32)
    seg = _seg_matrix(HEADS, HIM)
    asrc = jnp.dot(h * asrc_ref[...], seg, preferred_element_type=jnp.float32)
    adst = jnp.dot(h * adst_ref[...], seg, preferred_element_type=jnp.float32)
    bsum = (jnp.max(asrc, axis=0, keepdims=True)
            + jnp.max(adst, axis=0, keepdims=True))
    bvec_ref[...] = jnp.concatenate(
        [bsum, jnp.zeros((1, 8), jnp.float32)], axis=1)
    aug_ref[...] = jnp.concatenate(
        [h, asrc, jnp.full((N, 8), NEG, jnp.float32)], axis=1)
    dstt = jnp.concatenate([adst, jnp.full((N, 8), NEG, jnp.float32)], axis=1)
    dstt_ref[...] = jnp.concatenate(
        [dstt, jnp.full((NPAD - N, 16), NEG, jnp.float32)], axis=0)


_prep1 = pl.pallas_call(
    _prep1_body,
    out_shape=[
        jax.ShapeDtypeStruct((N, 144), jnp.float32),
        jax.ShapeDtypeStruct((NPAD, 16), jnp.float32),
        jax.ShapeDtypeStruct((1, 16), jnp.float32),
    ],
)


def _prep2_body(parts_ref, b1_ref, w2_ref, asrc2_ref, adst2_ref,
                aug_ref, dstt_ref, bvec_ref):
    aug1 = parts_ref[0] + parts_ref[1]
    msg = aug1[:N, 0:128]
    wsum = aug1[:N, 128:136]
    denf = jnp.dot(wsum, _seg_matrix(HEADS, HIM).T,
                   preferred_element_type=jnp.float32)
    x2 = jnp.maximum(msg / denf + b1_ref[...], 0.0)
    h2 = jnp.dot(x2, w2_ref[...], preferred_element_type=jnp.float32)

    lane0 = (lax.broadcasted_iota(jnp.int32, (1, 16), 1) == 0)
    a_s = asrc2_ref[...].T * lane0.astype(jnp.float32)
    a_d = adst2_ref[...].T * lane0.astype(jnp.float32)
    att_s = jnp.dot(h2, a_s, preferred_element_type=jnp.float32)
    att_d = jnp.dot(h2, a_d, preferred_element_type=jnp.float32)
    bsum = (jnp.max(att_s[:, 0:1], axis=0, keepdims=True)
            + jnp.max(att_d[:, 0:1], axis=0, keepdims=True))
    bvec_ref[...] = jnp.where(lane0, bsum, 0.0)
    att_s = jnp.where(lane0, att_s, NEG)
    att_d = jnp.where(lane0, att_d, NEG)
    aug_ref[...] = jnp.concatenate([h2, att_s], axis=1)
    dstt_ref[...] = jnp.concatenate(
        [att_d, jnp.full((NPAD - N, 16), NEG, jnp.float32)], axis=0)


_prep2 = pl.pallas_call(
    _prep2_body,
    out_shape=[
        jax.ShapeDtypeStruct((N, 80), jnp.float32),
        jax.ShapeDtypeStruct((NPAD, 16), jnp.float32),
        jax.ShapeDtypeStruct((1, 16), jnp.float32),
    ],
)


def _final_body(parts_ref, b2_ref, out_ref):
    aug2 = parts_ref[0] + parts_ref[1]
    msg = aug2[:N, 0:64]
    den = aug2[:N, 64:65]
    out_ref[...] = msg / den + b2_ref[...]


_final = pl.pallas_call(
    _final_body,
    out_shape=jax.ShapeDtypeStruct((N, OUT), jnp.float32),
)


def _edge_pass_body(row, hc, heads, ch,
                    aug_hbm, dstt_hbm, edges_hbm, bvec_hbm, out_hbm,
                    idx_a, idx_b, idx_c, rows_a, rows_b, rows_c,
                    drows_a, drows_b, drows_c,
                    bvec, acc, sem_a1, sem_a2, sem_b1, sem_b2,
                    sem_c1, sem_c2, sem_ia, sem_ib, sem_ic):
    cid = lax.axis_index("c")
    sid = lax.axis_index("s")
    wid = sid * NCORES + cid

    def _zero_row(i, _):
        for j in range(row // 16):
            rows_a[i, pl.ds(j * 16, 16)] = jnp.zeros((16,), jnp.float32)
        return 0
    lax.fori_loop(0, K, _zero_row, 0)
    base_row = sid * RPW
    for t in range(RPW // K):
        pltpu.sync_copy(rows_a, acc.at[pl.ds(base_row + t * K, K)])
    rem = RPW % K
    if rem:
        pltpu.sync_copy(rows_a.at[pl.ds(0, rem)],
                        acc.at[pl.ds(base_row + (RPW // K) * K, rem)])
    plsc.subcore_barrier()

    pltpu.sync_copy(bvec_hbm, bvec)
    base_c = wid * CPW

    bufs = ((idx_a, rows_a, drows_a, sem_a1, sem_a2, sem_ia),
            (idx_b, rows_b, drows_b, sem_b1, sem_b2, sem_ib),
            (idx_c, rows_c, drows_c, sem_c1, sem_c2, sem_ic))

    def _issue_gather(b):
        idx, rows, drows, s1, s2, _ = b
        pltpu.async_copy(aug_hbm.at[idx.at[0]], rows, s1)
        pltpu.async_copy(dstt_hbm.at[idx.at[1]], drows, s2)

    def _wait_gather(b):
        idx, rows, drows, s1, s2, _ = b
        pltpu.make_async_copy(aug_hbm.at[idx.at[0]], rows, s1).wait()
        pltpu.make_async_copy(dstt_hbm.at[idx.at[1]], drows, s2).wait()

    def _wait_idx(b):
        pltpu.make_async_copy(edges_hbm.at[0], b[0], b[5]).wait()

    def _compute_scatter(b):
        idx, rows, drows, _, _, _ = b
        bv = bvec[...]

        @plsc.parallel_loop(0, K, 1, unroll=4)
        def _edge(i):
            ev = rows[i, pl.ds(hc, 16)] + drows[i, :]
            ev = jnp.where(ev >= 0.0, ev, ev * NEG_SLOPE)
            wv = jnp.exp(ev - bv)
            rows[i, pl.ds(hc, 16)] = wv
            for j in range(heads):
                ws = wv[j]
                for v in range(ch // 16):
                    off = j * ch + v * 16
                    rows[i, pl.ds(off, 16)] = rows[i, pl.ds(off, 16)] * ws
        pltpu.sync_copy(rows, acc.at[idx.at[1]], add=True)

    pltpu.sync_copy(edges_hbm.at[base_c], idx_a)
    pltpu.sync_copy(edges_hbm.at[base_c + 1], idx_b)
    pltpu.async_copy(edges_hbm.at[base_c + 2], idx_c, sem_ic)
    _issue_gather(bufs[0])
    _issue_gather(bufs[1])

    def _step(c, cur, nxt, nn):
        _wait_idx(nn)
        _issue_gather(nn)
        _wait_gather(cur)
        _compute_scatter(cur)
        pltpu.async_copy(edges_hbm.at[c + 3], cur[0], cur[5])

    def _triple(r, _):
        c = base_c + r * 3
        _step(c, bufs[0], bufs[1], bufs[2])
        _step(c + 1, bufs[1], bufs[2], bufs[0])
        _step(c + 2, bufs[2], bufs[0], bufs[1])
        return 0
    lax.fori_loop(0, CPW // 3, _triple, 0)
    _wait_gather(bufs[0])
    _wait_gather(bufs[1])
    _wait_idx(bufs[2])

    plsc.subcore_barrier()
    pltpu.sync_copy(acc.at[pl.ds(base_row, RPW)],
                    out_hbm.at[cid].at[pl.ds(base_row, RPW)])


def _make_edge_pass(row, hc, heads, ch):
    return functools.partial(
        pl.kernel,
        out_type=jax.ShapeDtypeStruct((NCORES, NPAD, row), jnp.float32),
        mesh=plsc.VectorSubcoreMesh(core_axis_name="c", subcore_axis_name="s"),
        compiler_params=pltpu.CompilerParams(use_tc_tiling_on_sc=False),
        scratch_types=(
            [pltpu.VMEM((2, K), jnp.int32)] * 3
            + [pltpu.VMEM((K, row), jnp.float32)] * 3
            + [pltpu.VMEM((K, 16), jnp.float32)] * 3
            + [pltpu.VMEM((16,), jnp.float32),
               pltpu.VMEM_SHARED((NPAD, row), jnp.float32)]
            + [pltpu.SemaphoreType.DMA] * 9
        ),
    )(functools.partial(_edge_pass_body, row, hc, heads, ch))


_edge_pass1 = _make_edge_pass(144, 128, HEADS, HIM)
_edge_pass2 = _make_edge_pass(80, 64, 1, OUT)


def kernel(x, edge_index, W1, a_src1, a_dst1, b1, W2, a_src2, a_dst2, b2):
    loops = jnp.arange(N, dtype=jnp.int32)
    npad_e = EPAD + 3 * K - (E + N)
    src = jnp.concatenate(
        [edge_index[0], loops,
         jnp.zeros((npad_e,), jnp.int32)]).reshape(-1, K)
    dst = jnp.concatenate(
        [edge_index[1], loops,
         jnp.full((npad_e,), N, jnp.int32)]).reshape(-1, K)
    edges = jnp.stack([src, dst], axis=1)

    aug1, dstt1, bvec1 = _prep1(
        x, W1, a_src1.reshape(1, HEADS * HIM), a_dst1.reshape(1, HEADS * HIM))
    parts1 = _edge_pass1(aug1, dstt1, edges, bvec1.reshape(16))

    aug2, dstt2, bvec2 = _prep2(
        parts1, b1.reshape(1, HEADS * HIM), W2, a_src2, a_dst2)
    parts2 = _edge_pass2(aug2, dstt2, edges, bvec2.reshape(16))

    return _final(parts2, b2.reshape(1, OUT))


# speedup vs baseline: 1.1529x; 1.0006x over previous
import functools

import jax
import jax.numpy as jnp
from jax import lax
from jax.experimental import pallas as pl
from jax.experimental.pallas import tpu as pltpu
from jax.experimental.pallas import tpu_sc as plsc

N = 10000
E = 320000
D = 128
HIM = 16
HEADS = 8
OUT = 64
NEG_SLOPE = 0.2

NPAD = 10016
NCORES = 2
NSUB = 16
NW = NCORES * NSUB
K = 80
EPAD = 330240
CPW = EPAD // (NW * K)
RPW = NPAD // NSUB
ZR = 8
NEG = -1e30


def _seg_matrix(heads, ch):
    r = lax.broadcasted_iota(jnp.int32, (heads * ch, heads), 0) // ch
    c = lax.broadcasted_iota(jnp.int32, (heads * ch, heads), 1)
    return (r == c).astype(jnp.float32)


def _prep1_body(x_ref, w_ref, asrc_ref, adst_ref, aug_ref, dstt_ref, bvec_ref):
    h = jnp.dot(x_ref[...], w_ref[...], preferred_element_type=jnp.float32)
    seg = _seg_matrix(HEADS, HIM)
    asrc = jnp.dot(h * asrc_ref[...], seg, preferred_element_type=jnp.float32)
    adst = jnp.dot(h * adst_ref[...], seg, preferred_element_type=jnp.float32)
    bsum = (jnp.max(asrc, axis=0, keepdims=True)
            + jnp.max(adst, axis=0, keepdims=True))
    bvec_ref[...] = jnp.concatenate(
        [bsum, jnp.zeros((1, 8), jnp.float32)], axis=1)
    aug_ref[...] = jnp.concatenate(
        [h, asrc, jnp.full((N, 8), NEG, jnp.float32)], axis=1)
    dstt = jnp.concatenate([adst, jnp.full((N, 8), NEG, jnp.float32)], axis=1)
    dstt_ref[...] = jnp.concatenate(
        [dstt, jnp.full((NPAD - N, 16), NEG, jnp.float32)], axis=0)


_prep1 = pl.pallas_call(
    _prep1_body,
    out_shape=[
        jax.ShapeDtypeStruct((N, 144), jnp.float32),
        jax.ShapeDtypeStruct((NPAD, 16), jnp.float32),
        jax.ShapeDtypeStruct((1, 16), jnp.float32),
    ],
)


def _prep2_body(parts_ref, b1_ref, w2_ref, asrc2_ref, adst2_ref,
                aug_ref, dstt_ref, bvec_ref):
    aug1 = parts_ref[0] + parts_ref[1]
    msg = aug1[:N, 0:128]
    wsum = aug1[:N, 128:136]
    denf = jnp.dot(wsum, _seg_matrix(HEADS, HIM).T,
                   preferred_element_type=jnp.float32)
    x2 = jnp.maximum(msg / denf + b1_ref[...], 0.0)
    h2 = jnp.dot(x2, w2_ref[...], preferred_element_type=jnp.float32)

    lane0 = (lax.broadcasted_iota(jnp.int32, (1, 16), 1) == 0)
    a_s = asrc2_ref[...].T * lane0.astype(jnp.float32)
    a_d = adst2_ref[...].T * lane0.astype(jnp.float32)
    att_s = jnp.dot(h2, a_s, preferred_element_type=jnp.float32)
    att_d = jnp.dot(h2, a_d, preferred_element_type=jnp.float32)
    bsum = (jnp.max(att_s[:, 0:1], axis=0, keepdims=True)
            + jnp.max(att_d[:, 0:1], axis=0, keepdims=True))
    bvec_ref[...] = jnp.where(lane0, bsum, 0.0)
    att_s = jnp.where(lane0, att_s, NEG)
    att_d = jnp.where(lane0, att_d, NEG)
    aug_ref[...] = jnp.concatenate([h2, att_s], axis=1)
    dstt_ref[...] = jnp.concatenate(
        [att_d, jnp.full((NPAD - N, 16), NEG, jnp.float32)], axis=0)


_prep2 = pl.pallas_call(
    _prep2_body,
    out_shape=[
        jax.ShapeDtypeStruct((N, 80), jnp.float32),
        jax.ShapeDtypeStruct((NPAD, 16), jnp.float32),
        jax.ShapeDtypeStruct((1, 16), jnp.float32),
    ],
)


def _final_body(parts_ref, b2_ref, out_ref):
    aug2 = parts_ref[0] + parts_ref[1]
    msg = aug2[:N, 0:64]
    den = aug2[:N, 64:65]
    out_ref[...] = msg / den + b2_ref[...]


_final = pl.pallas_call(
    _final_body,
    out_shape=jax.ShapeDtypeStruct((N, OUT), jnp.float32),
)


def _edge_pass_body(row, hc, heads, ch,
                    aug_hbm, dstt_hbm, edges_hbm, bvec_hbm, out_hbm,
                    idx_a, idx_b, idx_c, rows_a, rows_b, rows_c,
                    drows_a, drows_b, drows_c,
                    bvec, zrows, acc, sem_a1, sem_a2, sem_b1, sem_b2,
                    sem_c1, sem_c2, sem_ia, sem_ib, sem_ic, sem_z):
    cid = lax.axis_index("c")
    sid = lax.axis_index("s")
    wid = sid * NCORES + cid

    base_row = sid * RPW
    base_c = wid * CPW

    bufs = ((idx_a, rows_a, drows_a, sem_a1, sem_a2, sem_ia),
            (idx_b, rows_b, drows_b, sem_b1, sem_b2, sem_ib),
            (idx_c, rows_c, drows_c, sem_c1, sem_c2, sem_ic))

    def _issue_gather(b):
        idx, rows, drows, s1, s2, _ = b
        pltpu.async_copy(aug_hbm.at[idx.at[0]], rows, s1)
        pltpu.async_copy(dstt_hbm.at[idx.at[1]], drows, s2)

    def _wait_gather(b):
        idx, rows, drows, s1, s2, _ = b
        pltpu.make_async_copy(aug_hbm.at[idx.at[0]], rows, s1).wait()
        pltpu.make_async_copy(dstt_hbm.at[idx.at[1]], drows, s2).wait()

    def _wait_idx(b):
        pltpu.make_async_copy(edges_hbm.at[0], b[0], b[5]).wait()

    def _compute_scatter(b):
        idx, rows, drows, _, _, _ = b
        bv = bvec[...]

        @plsc.parallel_loop(0, K, 1, unroll=4)
        def _edge(i):
            ev = rows[i, pl.ds(hc, 16)] + drows[i, :]
            ev = jnp.where(ev >= 0.0, ev, ev * NEG_SLOPE)
            wv = jnp.exp(ev - bv)
            rows[i, pl.ds(hc, 16)] = wv
            for j in range(heads):
                ws = wv[j]
                for v in range(ch // 16):
                    off = j * ch + v * 16
                    rows[i, pl.ds(off, 16)] = rows[i, pl.ds(off, 16)] * ws
        pltpu.sync_copy(rows, acc.at[idx.at[1]], add=True)

    pltpu.sync_copy(edges_hbm.at[base_c], idx_a)
    pltpu.sync_copy(edges_hbm.at[base_c + 1], idx_b)
    pltpu.async_copy(edges_hbm.at[base_c + 2], idx_c, sem_ic)
    _issue_gather(bufs[0])
    _issue_gather(bufs[1])
    pltpu.sync_copy(bvec_hbm, bvec)

    def _zero_row(i, _):
        for j in range(row // 16):
            zrows[i, pl.ds(j * 16, 16)] = jnp.zeros((16,), jnp.float32)
        return 0
    lax.fori_loop(0, ZR, _zero_row, 0)
    nz = RPW // ZR
    for t in range(nz):
        pltpu.async_copy(zrows, acc.at[pl.ds(base_row + t * ZR, ZR)], sem_z)
    rem = RPW % ZR
    if rem:
        pltpu.async_copy(zrows.at[pl.ds(0, rem)],
                         acc.at[pl.ds(base_row + nz * ZR, rem)], sem_z)
    for t in range(nz):
        pltpu.make_async_copy(zrows, acc.at[pl.ds(base_row, ZR)], sem_z).wait()
    if rem:
        pltpu.make_async_copy(zrows.at[pl.ds(0, rem)],
                              acc.at[pl.ds(base_row, rem)], sem_z).wait()
    plsc.subcore_barrier()

    def _step(c, cur, nxt, nn):
        _wait_idx(nn)
        _issue_gather(nn)
        _wait_gather(cur)
        _compute_scatter(cur)
        pltpu.async_copy(edges_hbm.at[c + 3], cur[0], cur[5])

    def _triple(r, _):
        c = base_c + r * 3
        _step(c, bufs[0], bufs[1], bufs[2])
        _step(c + 1, bufs[1], bufs[2], bufs[0])
        _step(c + 2, bufs[2], bufs[0], bufs[1])
        return 0
    lax.fori_loop(0, CPW // 3, _triple, 0)
    _wait_gather(bufs[0])
    _wait_gather(bufs[1])
    _wait_idx(bufs[2])

    plsc.subcore_barrier()
    pltpu.sync_copy(acc.at[pl.ds(base_row, RPW)],
                    out_hbm.at[cid].at[pl.ds(base_row, RPW)])


def _make_edge_pass(row, hc, heads, ch):
    return functools.partial(
        pl.kernel,
        out_type=jax.ShapeDtypeStruct((NCORES, NPAD, row), jnp.float32),
        mesh=plsc.VectorSubcoreMesh(core_axis_name="c", subcore_axis_name="s"),
        compiler_params=pltpu.CompilerParams(use_tc_tiling_on_sc=False),
        scratch_types=(
            [pltpu.VMEM((2, K), jnp.int32)] * 3
            + [pltpu.VMEM((K, row), jnp.float32)] * 3
            + [pltpu.VMEM((K, 16), jnp.float32)] * 3
            + [pltpu.VMEM((16,), jnp.float32),
               pltpu.VMEM((ZR, row), jnp.float32),
               pltpu.VMEM_SHARED((NPAD, row), jnp.float32)]
            + [pltpu.SemaphoreType.DMA] * 10
        ),
    )(functools.partial(_edge_pass_body, row, hc, heads, ch))


_edge_pass1 = _make_edge_pass(144, 128, HEADS, HIM)
_edge_pass2 = _make_edge_pass(80, 64, 1, OUT)


def kernel(x, edge_index, W1, a_src1, a_dst1, b1, W2, a_src2, a_dst2, b2):
    loops = jnp.arange(N, dtype=jnp.int32)
    npad_e = EPAD + 3 * K - (E + N)
    src = jnp.concatenate(
        [edge_index[0], loops,
         jnp.zeros((npad_e,), jnp.int32)]).reshape(-1, K)
    dst = jnp.concatenate(
        [edge_index[1], loops,
         jnp.full((npad_e,), N, jnp.int32)]).reshape(-1, K)
    edges = jnp.stack([src, dst], axis=1)

    aug1, dstt1, bvec1 = _prep1(
        x, W1, a_src1.reshape(1, HEADS * HIM), a_dst1.reshape(1, HEADS * HIM))
    parts1 = _edge_pass1(aug1, dstt1, edges, bvec1.reshape(16))

    aug2, dstt2, bvec2 = _prep2(
        parts1, b1.reshape(1, HEADS * HIM), W2, a_src2, a_dst2)
    parts2 = _edge_pass2(aug2, dstt2, edges, bvec2.reshape(16))

    return _final(parts2, b2.reshape(1, OUT))


# speedup vs baseline: 1.1547x; 1.0016x over previous
import functools

import jax
import jax.numpy as jnp
from jax import lax
from jax.experimental import pallas as pl
from jax.experimental.pallas import tpu as pltpu
from jax.experimental.pallas import tpu_sc as plsc

N = 10000
E = 320000
D = 128
HIM = 16
HEADS = 8
OUT = 64
NEG_SLOPE = 0.2

NPAD = 10016
NCORES = 2
NSUB = 16
NW = NCORES * NSUB
K = 80
EPAD = 330240
CPW = EPAD // (NW * K)
RPW = NPAD // NSUB
ZR = 8
NEG = -1e30


def _seg_matrix(heads, ch):
    r = lax.broadcasted_iota(jnp.int32, (heads * ch, heads), 0) // ch
    c = lax.broadcasted_iota(jnp.int32, (heads * ch, heads), 1)
    return (r == c).astype(jnp.float32)


def _interleave_perm(h, c_of_elem):
    nch = h.shape[1]
    ci = lax.broadcasted_iota(jnp.int32, (nch, nch), 0)
    ei = lax.broadcasted_iota(jnp.int32, (nch, nch), 1)
    perm = (ci == c_of_elem(ei)).astype(jnp.float32)
    return jnp.dot(h, perm, preferred_element_type=jnp.float32).astype(
        jnp.bfloat16)


def _prep1_body(x_ref, w_ref, asrc_ref, adst_ref,
                aug_ref, satt_ref, dstt_ref, bvec_ref):
    h = jnp.dot(x_ref[...], w_ref[...], preferred_element_type=jnp.float32)
    seg = _seg_matrix(HEADS, HIM)
    asrc = jnp.dot(h * asrc_ref[...], seg, preferred_element_type=jnp.float32)
    adst = jnp.dot(h * adst_ref[...], seg, preferred_element_type=jnp.float32)
    bsum = (jnp.max(asrc, axis=0, keepdims=True)
            + jnp.max(adst, axis=0, keepdims=True))
    bvec_ref[...] = jnp.concatenate(
        [bsum, jnp.zeros((1, 8), jnp.float32)], axis=1)
    aug_ref[...] = _interleave_perm(
        h, lambda e: 32 * (e // 32) + (e // 2) % 16 + 16 * (e % 2))
    satt_ref[...] = jnp.concatenate(
        [asrc, jnp.full((N, 8), NEG, jnp.float32)], axis=1)
    dstt = jnp.concatenate([adst, jnp.full((N, 8), NEG, jnp.float32)], axis=1)
    dstt_ref[...] = jnp.concatenate(
        [dstt, jnp.full((NPAD - N, 16), NEG, jnp.float32)], axis=0)


_prep1 = pl.pallas_call(
    _prep1_body,
    out_shape=[
        jax.ShapeDtypeStruct((N, 128), jnp.bfloat16),
        jax.ShapeDtypeStruct((N, 16), jnp.float32),
        jax.ShapeDtypeStruct((NPAD, 16), jnp.float32),
        jax.ShapeDtypeStruct((1, 16), jnp.float32),
    ],
)


def _prep2_body(parts_ref, b1_ref, w2_ref, asrc2_ref, adst2_ref,
                aug_ref, satt_ref, dstt_ref, bvec_ref):
    aug1 = parts_ref[0] + parts_ref[1]
    msg = aug1[:N, 0:128]
    wsum = aug1[:N, 128:136]
    denf = jnp.dot(wsum, _seg_matrix(HEADS, HIM).T,
                   preferred_element_type=jnp.float32)
    x2 = jnp.maximum(msg / denf + b1_ref[...], 0.0)
    h2 = jnp.dot(x2, w2_ref[...], preferred_element_type=jnp.float32)

    lane0 = (lax.broadcasted_iota(jnp.int32, (1, 16), 1) == 0)
    a_s = asrc2_ref[...].T * lane0.astype(jnp.float32)
    a_d = adst2_ref[...].T * lane0.astype(jnp.float32)
    att_s = jnp.dot(h2, a_s, preferred_element_type=jnp.float32)
    att_d = jnp.dot(h2, a_d, preferred_element_type=jnp.float32)
    bsum = (jnp.max(att_s[:, 0:1], axis=0, keepdims=True)
            + jnp.max(att_d[:, 0:1], axis=0, keepdims=True))
    bvec_ref[...] = jnp.where(lane0, bsum, 0.0)
    att_s = jnp.where(lane0, att_s, NEG)
    att_d = jnp.where(lane0, att_d, NEG)
    aug_ref[...] = _interleave_perm(h2, lambda e: e // 2 + 32 * (e % 2))
    satt_ref[...] = att_s
    dstt_ref[...] = jnp.concatenate(
        [att_d, jnp.full((NPAD - N, 16), NEG, jnp.float32)], axis=0)


_prep2 = pl.pallas_call(
    _prep2_body,
    out_shape=[
        jax.ShapeDtypeStruct((N, 64), jnp.bfloat16),
        jax.ShapeDtypeStruct((N, 16), jnp.float32),
        jax.ShapeDtypeStruct((NPAD, 16), jnp.float32),
        jax.ShapeDtypeStruct((1, 16), jnp.float32),
    ],
)


def _final_body(parts_ref, b2_ref, out_ref):
    aug2 = parts_ref[0] + parts_ref[1]
    msg = aug2[:N, 0:64]
    den = aug2[:N, 64:65]
    out_ref[...] = msg / den + b2_ref[...]


_final = pl.pallas_call(
    _final_body,
    out_shape=jax.ShapeDtypeStruct((N, OUT), jnp.float32),
)


def _edge_pass_body(srow, hc, heads,
                    aug_hbm, satt_hbm, dstt_hbm, edges_hbm, bvec_hbm,
                    out_hbm,
                    idx_a, idx_b, idx_c, rows_a, rows_b, rows_c,
                    satt_a, satt_b, satt_c, datt_a, datt_b, datt_c, rows_s,
                    bvec, zrows, acc, sem_a, sem_b, sem_c,
                    sem_ia, sem_ib, sem_ic, sem_z):
    cid = lax.axis_index("c")
    sid = lax.axis_index("s")
    wid = sid * NCORES + cid

    base_row = sid * RPW
    base_c = wid * CPW

    bufs = ((idx_a, rows_a, satt_a, datt_a, sem_a, sem_ia),
            (idx_b, rows_b, satt_b, datt_b, sem_b, sem_ib),
            (idx_c, rows_c, satt_c, datt_c, sem_c, sem_ic))

    def _issue_gather(b):
        idx, rows, satt, datt, s, _ = b
        pltpu.async_copy(aug_hbm.at[idx.at[0]], rows, s)
        pltpu.async_copy(satt_hbm.at[idx.at[0]], satt, s)
        pltpu.async_copy(dstt_hbm.at[idx.at[1]], datt, s)

    def _wait_gather(b):
        idx, rows, satt, datt, s, _ = b
        pltpu.make_async_copy(aug_hbm.at[idx.at[0]], rows, s).wait()
        pltpu.make_async_copy(satt_hbm.at[idx.at[0]], satt, s).wait()
        pltpu.make_async_copy(dstt_hbm.at[idx.at[1]], datt, s).wait()

    def _wait_idx(b):
        pltpu.make_async_copy(edges_hbm.at[0], b[0], b[5]).wait()

    def _compute_scatter(b):
        idx, rows, satt, datt, _, _ = b
        bv = bvec[...]

        @plsc.parallel_loop(0, K, 1, unroll=4)
        def _edge(i):
            ev = satt[i, :] + datt[i, :]
            ev = jnp.where(ev >= 0.0, ev, ev * NEG_SLOPE)
            wv = jnp.exp(ev - bv)
            rows_s[i, pl.ds(hc, 16)] = wv
            for p in range(hc // 32):
                wb = rows[i, pl.ds(32 * p, 32)]
                lo, hi = plsc.unpack(wb, format=plsc.PackFormat.INTERLEAVED)
                if heads > 1:
                    rows_s[i, pl.ds(32 * p, 16)] = lo * wv[2 * p]
                    rows_s[i, pl.ds(32 * p + 16, 16)] = hi * wv[2 * p + 1]
                else:
                    rows_s[i, pl.ds(16 * p, 16)] = lo * wv[0]
                    rows_s[i, pl.ds(hc // 2 + 16 * p, 16)] = hi * wv[0]
        pltpu.sync_copy(rows_s, acc.at[idx.at[1]], add=True)

    pltpu.sync_copy(edges_hbm.at[base_c], idx_a)
    pltpu.sync_copy(edges_hbm.at[base_c + 1], idx_b)
    pltpu.async_copy(edges_hbm.at[base_c + 2], idx_c, sem_ic)
    _issue_gather(bufs[0])
    _issue_gather(bufs[1])
    pltpu.sync_copy(bvec_hbm, bvec)

    def _zero_row(i, _):
        for j in range(srow // 16):
            zrows[i, pl.ds(j * 16, 16)] = jnp.zeros((16,), jnp.float32)
        return 0
    lax.fori_loop(0, ZR, _zero_row, 0)
    nz = RPW // ZR
    for t in range(nz):
        pltpu.async_copy(zrows, acc.at[pl.ds(base_row + t * ZR, ZR)], sem_z)
    rem = RPW % ZR
    if rem:
        pltpu.async_copy(zrows.at[pl.ds(0, rem)],
                         acc.at[pl.ds(base_row + nz * ZR, rem)], sem_z)
    for t in range(nz):
        pltpu.make_async_copy(zrows, acc.at[pl.ds(base_row, ZR)], sem_z).wait()
    if rem:
        pltpu.make_async_copy(zrows.at[pl.ds(0, rem)],
                              acc.at[pl.ds(base_row, rem)], sem_z).wait()
    plsc.subcore_barrier()

    def _step(c, cur, nxt, nn):
        _wait_idx(nn)
        _issue_gather(nn)
        _wait_gather(cur)
        _compute_scatter(cur)
        pltpu.async_copy(edges_hbm.at[c + 3], cur[0], cur[5])

    def _triple(r, _):
        c = base_c + r * 3
        _step(c, bufs[0], bufs[1], bufs[2])
        _step(c + 1, bufs[1], bufs[2], bufs[0])
        _step(c + 2, bufs[2], bufs[0], bufs[1])
        return 0
    lax.fori_loop(0, CPW // 3, _triple, 0)
    _wait_gather(bufs[0])
    _wait_gather(bufs[1])
    _wait_idx(bufs[2])

    plsc.subcore_barrier()
    pltpu.sync_copy(acc.at[pl.ds(base_row, RPW)],
                    out_hbm.at[cid].at[pl.ds(base_row, RPW)])


def _make_edge_pass(srow, hc, heads):
    return functools.partial(
        pl.kernel,
        out_type=jax.ShapeDtypeStruct((NCORES, NPAD, srow), jnp.float32),
        mesh=plsc.VectorSubcoreMesh(core_axis_name="c", subcore_axis_name="s"),
        compiler_params=pltpu.CompilerParams(
            use_tc_tiling_on_sc=False, needs_layout_passes=False),
        scratch_types=(
            [pltpu.VMEM((2, K), jnp.int32)] * 3
            + [pltpu.VMEM((K, hc), jnp.bfloat16)] * 3
            + [pltpu.VMEM((K, 16), jnp.float32)] * 6
            + [pltpu.VMEM((K, srow), jnp.float32),
               pltpu.VMEM((16,), jnp.float32),
               pltpu.VMEM((ZR, srow), jnp.float32),
               pltpu.VMEM_SHARED((NPAD, srow), jnp.float32)]
            + [pltpu.SemaphoreType.DMA] * 7
        ),
    )(functools.partial(_edge_pass_body, srow, hc, heads))


_edge_pass1 = _make_edge_pass(144, 128, HEADS)
_edge_pass2 = _make_edge_pass(80, 64, 1)


def kernel(x, edge_index, W1, a_src1, a_dst1, b1, W2, a_src2, a_dst2, b2):
    loops = jnp.arange(N, dtype=jnp.int32)
    npad_e = EPAD + 3 * K - (E + N)
    src = jnp.concatenate(
        [edge_index[0], loops,
         jnp.zeros((npad_e,), jnp.int32)]).reshape(-1, K)
    dst = jnp.concatenate(
        [edge_index[1], loops,
         jnp.full((npad_e,), N, jnp.int32)]).reshape(-1, K)
    edges = jnp.stack([src, dst], axis=1)

    aug1, satt1, dstt1, bvec1 = _prep1(
        x, W1, a_src1.reshape(1, HEADS * HIM), a_dst1.reshape(1, HEADS * HIM))
    parts1 = _edge_pass1(aug1, satt1, dstt1, edges, bvec1.reshape(16))

    aug2, satt2, dstt2, bvec2 = _prep2(
        parts1, b1.reshape(1, HEADS * HIM), W2, a_src2, a_dst2)
    parts2 = _edge_pass2(aug2, satt2, dstt2, edges, bvec2.reshape(16))

    return _final(parts2, b2.reshape(1, OUT))


# speedup vs baseline: 1.3791x; 1.1944x over previous
import functools

import jax
import jax.numpy as jnp
from jax import lax
from jax.experimental import pallas as pl
from jax.experimental.pallas import tpu as pltpu
from jax.experimental.pallas import tpu_sc as plsc

N = 10000
E = 320000
D = 128
HIM = 16
HEADS = 8
OUT = 64
NEG_SLOPE = 0.2

NPAD = 10016
NCORES = 2
NSUB = 16
NW = NCORES * NSUB
K = 80
EPAD = 330240
CPW = EPAD // (NW * K)
RPW = NPAD // NSUB
ZR = 8
NEG = -1e30


def _seg_matrix(heads, ch):
    r = lax.broadcasted_iota(jnp.int32, (heads * ch, heads), 0) // ch
    c = lax.broadcasted_iota(jnp.int32, (heads * ch, heads), 1)
    return (r == c).astype(jnp.float32)


def _interleave_perm(h, c_of_elem):
    nch = h.shape[1]
    ci = lax.broadcasted_iota(jnp.int32, (nch, nch), 0)
    ei = lax.broadcasted_iota(jnp.int32, (nch, nch), 1)
    perm = (ci == c_of_elem(ei)).astype(jnp.float32)
    return jnp.dot(h, perm, preferred_element_type=jnp.float32).astype(
        jnp.bfloat16)


def _prep1_body(x_ref, w_ref, asrc_ref, adst_ref,
                aug_ref, satt_ref, dstt_ref, bvec_ref):
    h = jnp.dot(x_ref[...], w_ref[...], preferred_element_type=jnp.float32)
    seg = _seg_matrix(HEADS, HIM)
    asrc = jnp.dot(h * asrc_ref[...], seg, preferred_element_type=jnp.float32)
    adst = jnp.dot(h * adst_ref[...], seg, preferred_element_type=jnp.float32)
    bsum = (jnp.max(asrc, axis=0, keepdims=True)
            + jnp.max(adst, axis=0, keepdims=True))
    bvec_ref[...] = jnp.concatenate(
        [bsum, jnp.zeros((1, 8), jnp.float32)], axis=1)
    aug_ref[...] = _interleave_perm(
        h, lambda e: 32 * (e // 32) + (e // 2) % 16 + 16 * (e % 2))
    satt_ref[...] = jnp.concatenate(
        [asrc, jnp.full((N, 8), NEG, jnp.float32)], axis=1)
    dstt = jnp.concatenate([adst, jnp.full((N, 8), NEG, jnp.float32)], axis=1)
    dstt_ref[...] = jnp.concatenate(
        [dstt, jnp.full((NPAD - N, 16), NEG, jnp.float32)], axis=0)


_prep1 = pl.pallas_call(
    _prep1_body,
    out_shape=[
        jax.ShapeDtypeStruct((N, 128), jnp.bfloat16),
        jax.ShapeDtypeStruct((N, 16), jnp.float32),
        jax.ShapeDtypeStruct((NPAD, 16), jnp.float32),
        jax.ShapeDtypeStruct((1, 16), jnp.float32),
    ],
)


def _prep2_body(parts_ref, b1_ref, w2_ref, asrc2_ref, adst2_ref,
                aug_ref, satt_ref, dstt_ref, bvec_ref):
    aug1 = parts_ref[0] + parts_ref[1]
    msg = aug1[:N, 0:128]
    wsum = aug1[:N, 128:136]
    denf = jnp.dot(wsum, _seg_matrix(HEADS, HIM).T,
                   preferred_element_type=jnp.float32)
    x2 = jnp.maximum(msg / denf + b1_ref[...], 0.0)
    h2 = jnp.dot(x2, w2_ref[...], preferred_element_type=jnp.float32)

    lane0 = (lax.broadcasted_iota(jnp.int32, (1, 16), 1) == 0)
    a_s = asrc2_ref[...].T * lane0.astype(jnp.float32)
    a_d = adst2_ref[...].T * lane0.astype(jnp.float32)
    att_s = jnp.dot(h2, a_s, preferred_element_type=jnp.float32)
    att_d = jnp.dot(h2, a_d, preferred_element_type=jnp.float32)
    bsum = (jnp.max(att_s[:, 0:1], axis=0, keepdims=True)
            + jnp.max(att_d[:, 0:1], axis=0, keepdims=True))
    bvec_ref[...] = jnp.where(lane0, bsum, 0.0)
    att_s = jnp.where(lane0, att_s, NEG)
    att_d = jnp.where(lane0, att_d, NEG)
    aug_ref[...] = _interleave_perm(h2, lambda e: e // 2 + 32 * (e % 2))
    satt_ref[...] = att_s
    dstt_ref[...] = jnp.concatenate(
        [att_d, jnp.full((NPAD - N, 16), NEG, jnp.float32)], axis=0)


_prep2 = pl.pallas_call(
    _prep2_body,
    out_shape=[
        jax.ShapeDtypeStruct((N, 64), jnp.bfloat16),
        jax.ShapeDtypeStruct((N, 16), jnp.float32),
        jax.ShapeDtypeStruct((NPAD, 16), jnp.float32),
        jax.ShapeDtypeStruct((1, 16), jnp.float32),
    ],
)


def _final_body(parts_ref, b2_ref, out_ref):
    aug2 = parts_ref[0] + parts_ref[1]
    msg = aug2[:N, 0:64]
    den = aug2[:N, 64:65]
    out_ref[...] = msg / den + b2_ref[...]


_final = pl.pallas_call(
    _final_body,
    out_shape=jax.ShapeDtypeStruct((N, OUT), jnp.float32),
)


def _edge_pass_body(srow, hc, heads,
                    aug_hbm, satt_hbm, dstt_hbm, edges_hbm, bvec_hbm,
                    out_hbm,
                    idx_a, idx_b, idx_c, rows_a, rows_b, rows_c,
                    satt_a, satt_b, satt_c, datt_a, datt_b, datt_c, rows_s,
                    bvec, zrows, acc, sem_a, sem_b, sem_c,
                    sem_ia, sem_ib, sem_ic, sem_z, sem_s):
    cid = lax.axis_index("c")
    sid = lax.axis_index("s")
    wid = sid * NCORES + cid

    base_row = sid * RPW
    base_c = wid * CPW

    bufs = ((idx_a, rows_a, satt_a, datt_a, sem_a, sem_ia),
            (idx_b, rows_b, satt_b, datt_b, sem_b, sem_ib),
            (idx_c, rows_c, satt_c, datt_c, sem_c, sem_ic))

    def _issue_gather(b):
        idx, rows, satt, datt, s, _ = b
        pltpu.async_copy(aug_hbm.at[idx.at[0]], rows, s)
        pltpu.async_copy(satt_hbm.at[idx.at[0]], satt, s)
        pltpu.async_copy(dstt_hbm.at[idx.at[1]], datt, s)

    def _wait_gather(b):
        idx, rows, satt, datt, s, _ = b
        pltpu.make_async_copy(aug_hbm.at[idx.at[0]], rows, s).wait()
        pltpu.make_async_copy(satt_hbm.at[idx.at[0]], satt, s).wait()
        pltpu.make_async_copy(dstt_hbm.at[idx.at[1]], datt, s).wait()

    def _wait_idx(b):
        pltpu.make_async_copy(edges_hbm.at[0], b[0], b[5]).wait()

    def _wait_scatter():
        pltpu.make_async_copy(rows_s, acc.at[idx_a.at[1]], sem_s).wait()

    def _compute_scatter(b):
        idx, rows, satt, datt, _, _ = b
        bv = bvec[...]
        _wait_scatter()

        @plsc.parallel_loop(0, K, 1, unroll=4)
        def _edge(i):
            ev = satt[i, :] + datt[i, :]
            ev = jnp.where(ev >= 0.0, ev, ev * NEG_SLOPE)
            wv = jnp.exp(ev - bv)
            rows_s[i, pl.ds(hc, 16)] = wv
            for p in range(hc // 32):
                wb = rows[i, pl.ds(32 * p, 32)]
                lo, hi = plsc.unpack(wb, format=plsc.PackFormat.INTERLEAVED)
                if heads > 1:
                    rows_s[i, pl.ds(32 * p, 16)] = lo * wv[2 * p]
                    rows_s[i, pl.ds(32 * p + 16, 16)] = hi * wv[2 * p + 1]
                else:
                    rows_s[i, pl.ds(16 * p, 16)] = lo * wv[0]
                    rows_s[i, pl.ds(hc // 2 + 16 * p, 16)] = hi * wv[0]
        pltpu.async_copy(rows_s, acc.at[idx.at[1]], sem_s, add=True)

    pltpu.sync_copy(edges_hbm.at[base_c], idx_a)
    pltpu.sync_copy(edges_hbm.at[base_c + 1], idx_b)
    pltpu.async_copy(edges_hbm.at[base_c + 2], idx_c, sem_ic)
    _issue_gather(bufs[0])
    _issue_gather(bufs[1])
    pltpu.sync_copy(bvec_hbm, bvec)

    def _zero_row(i, _):
        for j in range(srow // 16):
            zrows[i, pl.ds(j * 16, 16)] = jnp.zeros((16,), jnp.float32)
        return 0
    lax.fori_loop(0, ZR, _zero_row, 0)
    nz = RPW // ZR
    for t in range(nz):
        pltpu.async_copy(zrows, acc.at[pl.ds(base_row + t * ZR, ZR)], sem_z)
    rem = RPW % ZR
    if rem:
        pltpu.async_copy(zrows.at[pl.ds(0, rem)],
                         acc.at[pl.ds(base_row + nz * ZR, rem)], sem_z)
    for t in range(nz):
        pltpu.make_async_copy(zrows, acc.at[pl.ds(base_row, ZR)], sem_z).wait()
    if rem:
        pltpu.make_async_copy(zrows.at[pl.ds(0, rem)],
                              acc.at[pl.ds(base_row, rem)], sem_z).wait()
    def _zero_srow(i, _):
        for j in range(srow // 16):
            rows_s[i, pl.ds(j * 16, 16)] = jnp.zeros((16,), jnp.float32)
        return 0
    lax.fori_loop(0, K, _zero_srow, 0)
    plsc.subcore_barrier()
    pltpu.async_copy(rows_s, acc.at[idx_a.at[1]], sem_s, add=True)

    def _step(c, cur, nxt, nn):
        _wait_idx(nn)
        _issue_gather(nn)
        _wait_gather(cur)
        _compute_scatter(cur)
        pltpu.async_copy(edges_hbm.at[c + 3], cur[0], cur[5])

    def _triple(r, _):
        c = base_c + r * 3
        _step(c, bufs[0], bufs[1], bufs[2])
        _step(c + 1, bufs[1], bufs[2], bufs[0])
        _step(c + 2, bufs[2], bufs[0], bufs[1])
        return 0
    lax.fori_loop(0, CPW // 3, _triple, 0)
    _wait_gather(bufs[0])
    _wait_gather(bufs[1])
    _wait_idx(bufs[2])
    _wait_scatter()

    plsc.subcore_barrier()
    pltpu.sync_copy(acc.at[pl.ds(base_row, RPW)],
                    out_hbm.at[cid].at[pl.ds(base_row, RPW)])


def _make_edge_pass(srow, hc, heads):
    return functools.partial(
        pl.kernel,
        out_type=jax.ShapeDtypeStruct((NCORES, NPAD, srow), jnp.float32),
        mesh=plsc.VectorSubcoreMesh(core_axis_name="c", subcore_axis_name="s"),
        compiler_params=pltpu.CompilerParams(
            use_tc_tiling_on_sc=False, needs_layout_passes=False),
        scratch_types=(
            [pltpu.VMEM((2, K), jnp.int32)] * 3
            + [pltpu.VMEM((K, hc), jnp.bfloat16)] * 3
            + [pltpu.VMEM((K, 16), jnp.float32)] * 6
            + [pltpu.VMEM((K, srow), jnp.float32),
               pltpu.VMEM((16,), jnp.float32),
               pltpu.VMEM((ZR, srow), jnp.float32),
               pltpu.VMEM_SHARED((NPAD, srow), jnp.float32)]
            + [pltpu.SemaphoreType.DMA] * 8
        ),
    )(functools.partial(_edge_pass_body, srow, hc, heads))


_edge_pass1 = _make_edge_pass(144, 128, HEADS)
_edge_pass2 = _make_edge_pass(80, 64, 1)


def kernel(x, edge_index, W1, a_src1, a_dst1, b1, W2, a_src2, a_dst2, b2):
    loops = jnp.arange(N, dtype=jnp.int32)
    npad_e = EPAD + 3 * K - (E + N)
    src = jnp.concatenate(
        [edge_index[0], loops,
         jnp.zeros((npad_e,), jnp.int32)]).reshape(-1, K)
    dst = jnp.concatenate(
        [edge_index[1], loops,
         jnp.full((npad_e,), N, jnp.int32)]).reshape(-1, K)
    edges = jnp.stack([src, dst], axis=1)

    aug1, satt1, dstt1, bvec1 = _prep1(
        x, W1, a_src1.reshape(1, HEADS * HIM), a_dst1.reshape(1, HEADS * HIM))
    parts1 = _edge_pass1(aug1, satt1, dstt1, edges, bvec1.reshape(16))

    aug2, satt2, dstt2, bvec2 = _prep2(
        parts1, b1.reshape(1, HEADS * HIM), W2, a_src2, a_dst2)
    parts2 = _edge_pass2(aug2, satt2, dstt2, edges, bvec2.reshape(16))

    return _final(parts2, b2.reshape(1, OUT))
